# baseline (speedup 1.0000x reference)
import jax
import jax.numpy as jnp
import numpy as np
from jax import lax
from jax.experimental import pallas as pl
from jax.experimental.pallas import tpu as pltpu

_B, _F, _S, _P, _E, _FF, _K = 64, 7, 512, 96, 8, 2048, 2
_N = _B * _F
_ALPHA = 10.0


def _moe_body(x_hbm, ti_hbm, gw_hbm, gb_hbm, w1_ref, b1_hbm, w2_hbm, b2_hbm,
              out_ref, loss_ref,
              x3, ti3, xf, tif, gws, gbs, b1s, b2s, w2s, gates_ref, acc_ref,
              sem_x, sem_ti, sem_gw, sem_gb, sem_b1, sem_b2, sem_w2):
    e = pl.program_id(0)

    @pl.when(e == 0)
    def _setup_and_gating():
        cp_w2 = pltpu.make_async_copy(w2_hbm, w2s, sem_w2)
        cp_x = pltpu.make_async_copy(x_hbm, x3, sem_x)
        cp_ti = pltpu.make_async_copy(ti_hbm, ti3, sem_ti)
        cp_gw = pltpu.make_async_copy(gw_hbm, gws, sem_gw)
        cp_gb = pltpu.make_async_copy(gb_hbm, gbs, sem_gb)
        cp_b1 = pltpu.make_async_copy(b1_hbm, b1s, sem_b1)
        cp_b2 = pltpu.make_async_copy(b2_hbm, b2s, sem_b2)
        cp_w2.start()
        cp_x.start()
        cp_ti.start()
        cp_gw.start()
        cp_gb.start()
        cp_b1.start()
        cp_b2.start()

        cp_x.wait()
        cp_ti.wait()
        for i in range(_F):
            xf[pl.ds(i * _B, _B), :] = x3[:, i, :]
            tif[pl.ds(i * _B, _B), :] = ti3[:, i, :]

        cp_gw.wait()
        cp_gb.wait()
        logits = jnp.dot(tif[...], gws[...],
                         preferred_element_type=jnp.float32) + gbs[...]
        m1 = jnp.max(logits, axis=1, keepdims=True)
        idx = lax.broadcasted_iota(jnp.int32, (_N, _E), 1)
        first_idx = jnp.min(jnp.where(logits == m1, idx, _E), axis=1,
                            keepdims=True)
        m2 = jnp.max(jnp.where(idx == first_idx, -jnp.inf, logits), axis=1,
                     keepdims=True)
        below_topk = logits < m2
        ex = jnp.exp(logits - m1)
        sm = ex / jnp.sum(ex, axis=1, keepdims=True)
        outv = jnp.where(below_topk, _ALPHA * jnp.log(sm + 1.0),
                         _ALPHA * (jnp.exp(sm) - 1.0))
        mo = jnp.max(outv, axis=1, keepdims=True)
        exo = jnp.exp(outv - mo)
        gates = exo / jnp.sum(exo, axis=1, keepdims=True)
        gates_ref[...] = gates

        row = lax.broadcasted_iota(jnp.int32, (_F, _N), 0)
        col = lax.broadcasted_iota(jnp.int32, (_F, _N), 1)
        sel = (col // _B == row).astype(jnp.float32)
        imp = jnp.dot(sel, gates, preferred_element_type=jnp.float32)
        mean = jnp.mean(imp, axis=1, keepdims=True)
        var = jnp.sum((imp - mean) ** 2, axis=1, keepdims=True) / (_E - 1)
        loss_ref[...] = jnp.sum(var / (mean ** 2 + 1e-10),
                                keepdims=True).reshape(1, 1)

        cp_b1.wait()
        cp_b2.wait()
        cp_w2.wait()

    erow = lax.broadcasted_iota(jnp.int32, (_E, 1), 0)
    b1row = jnp.sum(jnp.where(erow == e, b1s[...], 0.0), axis=0,
                    keepdims=True)
    b2row = jnp.sum(jnp.where(erow == e, b2s[...], 0.0), axis=0,
                    keepdims=True)

    xb = xf[...].astype(jnp.bfloat16)
    _C = 4
    _FC = _FF // _C
    o = b2row
    for c in range(_C):
        w1b = w1_ref[0, :, c * _FC:(c + 1) * _FC].astype(jnp.bfloat16)
        h = jnp.dot(xb, w1b, preferred_element_type=jnp.float32)
        h = h + b1row[:, c * _FC:(c + 1) * _FC]
        h = 0.5 * h * (1.0 + lax.erf(h * np.float32(1.0 / np.sqrt(2.0))))
        o = o + jnp.dot(h.astype(jnp.bfloat16),
                        w2s[e, c * _FC:(c + 1) * _FC, :].astype(jnp.bfloat16),
                        preferred_element_type=jnp.float32)

    lane = lax.broadcasted_iota(jnp.int32, (_N, _E), 1)
    g = jnp.sum(jnp.where(lane == e, gates_ref[...], 0.0), axis=1,
                keepdims=True)
    contrib = g * o

    @pl.when(e == 0)
    def _init():
        acc_ref[...] = contrib

    @pl.when(e > 0)
    def _acc():
        acc_ref[...] += contrib

    @pl.when(e == _E - 1)
    def _writeback():
        for i in range(_F):
            out_ref[:, i, :] = acc_ref[pl.ds(i * _B, _B), :]


def kernel(x, time_embedding, gate_W, gate_b, W1, b1, W2, b2):
    gb = gate_b.reshape(1, _E)

    out, loss = pl.pallas_call(
        _moe_body,
        grid=(_E,),
        in_specs=[
            pl.BlockSpec(memory_space=pltpu.MemorySpace.HBM),
            pl.BlockSpec(memory_space=pltpu.MemorySpace.HBM),
            pl.BlockSpec(memory_space=pltpu.MemorySpace.HBM),
            pl.BlockSpec(memory_space=pltpu.MemorySpace.HBM),
            pl.BlockSpec((1, _S, _FF), lambda e: (e, 0, 0)),
            pl.BlockSpec(memory_space=pltpu.MemorySpace.HBM),
            pl.BlockSpec(memory_space=pltpu.MemorySpace.HBM),
            pl.BlockSpec(memory_space=pltpu.MemorySpace.HBM),
        ],
        out_specs=[
            pl.BlockSpec((_B, _F, _P), lambda e: (0, 0, 0)),
            pl.BlockSpec((1, 1), lambda e: (0, 0)),
        ],
        out_shape=[
            jax.ShapeDtypeStruct((_B, _F, _P), jnp.float32),
            jax.ShapeDtypeStruct((1, 1), jnp.float32),
        ],
        scratch_shapes=[
            pltpu.VMEM((_B, _F, _S), jnp.float32),
            pltpu.VMEM((_B, _F, _S), jnp.float32),
            pltpu.VMEM((_N, _S), jnp.float32),
            pltpu.VMEM((_N, _S), jnp.float32),
            pltpu.VMEM((_S, _E), jnp.float32),
            pltpu.VMEM((1, _E), jnp.float32),
            pltpu.VMEM((_E, _FF), jnp.float32),
            pltpu.VMEM((_E, _P), jnp.float32),
            pltpu.VMEM((_E, _FF, _P), jnp.float32),
            pltpu.VMEM((_N, _E), jnp.float32),
            pltpu.VMEM((_N, _P), jnp.float32),
            pltpu.SemaphoreType.DMA,
            pltpu.SemaphoreType.DMA,
            pltpu.SemaphoreType.DMA,
            pltpu.SemaphoreType.DMA,
            pltpu.SemaphoreType.DMA,
            pltpu.SemaphoreType.DMA,
            pltpu.SemaphoreType.DMA,
        ],
        compiler_params=pltpu.CompilerParams(
            dimension_semantics=("arbitrary",),
            vmem_limit_bytes=61_000_000),
    )(x, time_embedding, gate_W, gb, W1, b1, W2, b2)

    return out, loss[0, 0]


# speedup vs baseline: 1.7319x; 1.7319x over previous
import jax
import jax.numpy as jnp
import numpy as np
from jax import lax
from jax.experimental import pallas as pl
from jax.experimental.pallas import tpu as pltpu

_B, _F, _S, _P, _E, _FF, _K = 64, 7, 512, 96, 8, 2048, 2
_N = _B * _F
_ALPHA = 10.0
_C = 4
_FC = _FF // _C


def _moe_body(x_hbm, ti_hbm, gw_hbm, gb_hbm, w1_ref, b1_hbm, w2_hbm, b2_hbm,
              out_ref, loss_ref,
              xf, tif, gws, gbs, b1s, b2s, w2s, gates_ref, acc_ref,
              sem_x, sem_ti, sem_gw, sem_gb, sem_b1, sem_b2, sem_w2):
    e = pl.program_id(0)

    @pl.when(e == 0)
    def _setup_and_gating():
        cp_w2 = pltpu.make_async_copy(w2_hbm, w2s, sem_w2)
        cp_gw = pltpu.make_async_copy(gw_hbm, gws, sem_gw)
        cp_gb = pltpu.make_async_copy(gb_hbm, gbs, sem_gb)
        cp_b1 = pltpu.make_async_copy(b1_hbm, b1s, sem_b1)
        cp_b2 = pltpu.make_async_copy(b2_hbm, b2s, sem_b2)
        cp_w2.start()
        cp_gw.start()
        cp_gb.start()
        cp_b1.start()
        cp_b2.start()
        cps_x = [pltpu.make_async_copy(
            x_hbm.at[i], xf.at[pl.ds(i * _B, _B), :], sem_x)
            for i in range(_F)]
        cps_ti = [pltpu.make_async_copy(
            ti_hbm.at[i], tif.at[pl.ds(i * _B, _B), :], sem_ti)
            for i in range(_F)]
        for cp in cps_x + cps_ti:
            cp.start()
        for cp in cps_x + cps_ti:
            cp.wait()

        cp_gw.wait()
        cp_gb.wait()
        logits = lax.dot_general(
            tif[...], gws[...], (((1,), (1,)), ((), ())),
            preferred_element_type=jnp.float32) + gbs[...]
        m1 = jnp.max(logits, axis=1, keepdims=True)
        idx = lax.broadcasted_iota(jnp.int32, (_N, _E), 1)
        first_idx = jnp.min(jnp.where(logits == m1, idx, _E), axis=1,
                            keepdims=True)
        m2 = jnp.max(jnp.where(idx == first_idx, -jnp.inf, logits), axis=1,
                     keepdims=True)
        below_topk = logits < m2
        ex = jnp.exp(logits - m1)
        sm = ex / jnp.sum(ex, axis=1, keepdims=True)
        outv = jnp.where(below_topk, _ALPHA * jnp.log(sm + 1.0),
                         _ALPHA * (jnp.exp(sm) - 1.0))
        mo = jnp.max(outv, axis=1, keepdims=True)
        exo = jnp.exp(outv - mo)
        gates = exo / jnp.sum(exo, axis=1, keepdims=True)
        gates_ref[...] = gates

        row = lax.broadcasted_iota(jnp.int32, (_F, _N), 0)
        col = lax.broadcasted_iota(jnp.int32, (_F, _N), 1)
        sel = (col // _B == row).astype(jnp.float32)
        imp = jnp.dot(sel, gates, preferred_element_type=jnp.float32)
        mean = jnp.mean(imp, axis=1, keepdims=True)
        var = jnp.sum((imp - mean) ** 2, axis=1, keepdims=True) / (_E - 1)
        loss_ref[...] = jnp.sum(var / (mean ** 2 + 1e-10),
                                keepdims=True).reshape(1, 1)

        cp_b1.wait()
        cp_b2.wait()
        cp_w2.wait()

    erow = lax.broadcasted_iota(jnp.int32, (_E, 1), 0)
    b1row = jnp.sum(jnp.where(erow == e, b1s[...], 0.0), axis=0,
                    keepdims=True)
    b2row = jnp.sum(jnp.where(erow == e, b2s[...], 0.0), axis=0,
                    keepdims=True)

    xb = xf[...].astype(jnp.bfloat16)
    o = b2row
    for c in range(_C):
        w1b = w1_ref[0, :, c * _FC:(c + 1) * _FC].astype(jnp.bfloat16)
        h = jnp.dot(xb, w1b, preferred_element_type=jnp.float32)
        h = h + b1row[:, c * _FC:(c + 1) * _FC]
        h = 0.5 * h * (1.0 + lax.erf(h * np.float32(1.0 / np.sqrt(2.0))))
        w2c = w2s[e, :, c * _FC:(c + 1) * _FC].astype(jnp.bfloat16)
        o = o + lax.dot_general(
            h.astype(jnp.bfloat16), w2c, (((1,), (1,)), ((), ())),
            preferred_element_type=jnp.float32)

    lane = lax.broadcasted_iota(jnp.int32, (_N, _E), 1)
    g = jnp.sum(jnp.where(lane == e, gates_ref[...], 0.0), axis=1,
                keepdims=True)
    contrib = g * o

    @pl.when(e == 0)
    def _init():
        acc_ref[...] = contrib

    @pl.when(e > 0)
    def _acc():
        acc_ref[...] += contrib

    @pl.when(e == _E - 1)
    def _writeback():
        for i in range(_F):
            out_ref[i] = acc_ref[pl.ds(i * _B, _B), :]


def kernel(x, time_embedding, gate_W, gate_b, W1, b1, W2, b2):
    xt = jnp.transpose(x, (1, 0, 2))
    tit = jnp.transpose(time_embedding, (1, 0, 2))
    gwt = gate_W.T
    w2t = jnp.transpose(W2, (0, 2, 1))
    gb = gate_b.reshape(1, _E)

    out, loss = pl.pallas_call(
        _moe_body,
        grid=(_E,),
        in_specs=[
            pl.BlockSpec(memory_space=pltpu.MemorySpace.HBM),
            pl.BlockSpec(memory_space=pltpu.MemorySpace.HBM),
            pl.BlockSpec(memory_space=pltpu.MemorySpace.HBM),
            pl.BlockSpec(memory_space=pltpu.MemorySpace.HBM),
            pl.BlockSpec((1, _S, _FF), lambda e: (e, 0, 0)),
            pl.BlockSpec(memory_space=pltpu.MemorySpace.HBM),
            pl.BlockSpec(memory_space=pltpu.MemorySpace.HBM),
            pl.BlockSpec(memory_space=pltpu.MemorySpace.HBM),
        ],
        out_specs=[
            pl.BlockSpec((_F, _B, _P), lambda e: (0, 0, 0)),
            pl.BlockSpec((1, 1), lambda e: (0, 0)),
        ],
        out_shape=[
            jax.ShapeDtypeStruct((_F, _B, _P), jnp.float32),
            jax.ShapeDtypeStruct((1, 1), jnp.float32),
        ],
        scratch_shapes=[
            pltpu.VMEM((_N, _S), jnp.float32),
            pltpu.VMEM((_N, _S), jnp.float32),
            pltpu.VMEM((_E, _S), jnp.float32),
            pltpu.VMEM((1, _E), jnp.float32),
            pltpu.VMEM((_E, _FF), jnp.float32),
            pltpu.VMEM((_E, _P), jnp.float32),
            pltpu.VMEM((_E, _P, _FF), jnp.float32),
            pltpu.VMEM((_N, _E), jnp.float32),
            pltpu.VMEM((_N, _P), jnp.float32),
            pltpu.SemaphoreType.DMA,
            pltpu.SemaphoreType.DMA,
            pltpu.SemaphoreType.DMA,
            pltpu.SemaphoreType.DMA,
            pltpu.SemaphoreType.DMA,
            pltpu.SemaphoreType.DMA,
            pltpu.SemaphoreType.DMA,
        ],
        compiler_params=pltpu.CompilerParams(
            dimension_semantics=("arbitrary",),
            vmem_limit_bytes=61_000_000),
    )(xt, tit, gwt, gb, W1, b1, w2t, b2)

    return jnp.transpose(out, (1, 0, 2)), loss[0, 0]
